# Initial kernel scaffold; baseline (speedup 1.0000x reference)
import jax
import jax.numpy as jnp
from jax.experimental import pallas as pl


def kernel(x, edge_index, edge_weight, W1, b1, W2, b2):
    raise NotImplementedError("write your pallas kernel here")


# speedup vs baseline: 2.1497x; 2.1497x over previous
import functools

import jax
import jax.numpy as jnp
from jax import lax
from jax.experimental import pallas as pl
from jax.experimental.pallas import tpu as pltpu
from jax.experimental.pallas import tpu_sc as plsc

N = 10000
E = 320000
C = 128
HID = 256
OUT = 128
NF = 6
NPOW = 5
POWERS = (1, 2, 4, 8, 16)
NC = 2
NS = 16
CH = C // NC
EPT = E // NS
CHUNK = 128
NFULL = EPT // CHUNK
REM = EPT - NFULL * CHUNK
ROWS_PT = N // NS
ZCH = 125
NLANE = 16


def _sc_spmm(xin, col, row, w):
    mesh = plsc.VectorSubcoreMesh(
        core_axis_name="c", subcore_axis_name="s",
        num_cores=NC, num_subcores=NS)

    @functools.partial(
        pl.kernel,
        out_type=jax.ShapeDtypeStruct((NPOW, NC, N, CH), jnp.float32),
        mesh=mesh,
        compiler_params=pltpu.CompilerParams(use_tc_tiling_on_sc=False),
        scratch_types=[
            pltpu.VMEM_SHARED((N, CH), jnp.float32),
            pltpu.VMEM_SHARED((N, CH), jnp.float32),
            pltpu.VMEM((ZCH, CH), jnp.float32),
            pltpu.VMEM((CHUNK,), jnp.int32),
            pltpu.VMEM((CHUNK,), jnp.int32),
            pltpu.VMEM((CHUNK,), jnp.float32),
            pltpu.VMEM((CHUNK, CH), jnp.float32),
            pltpu.VMEM((REM,), jnp.int32),
            pltpu.VMEM((REM,), jnp.int32),
            pltpu.VMEM((REM,), jnp.float32),
            pltpu.VMEM((REM, CH), jnp.float32),
        ],
    )
    def spmm_kernel(xin_h, col_h, row_h, w_h, out_h,
                    va, vb, zbuf, colv, rowv, wv, rows,
                    colv2, rowv2, wv2, rows2):
        c = lax.axis_index("c")
        s = lax.axis_index("s")
        r0 = s * ROWS_PT

        pltpu.sync_copy(xin_h.at[c, pl.ds(r0, ROWS_PT)],
                        va.at[pl.ds(r0, ROWS_PT)])

        @pl.loop(0, ZCH)
        def _(r):
            for j in range(CH // NLANE):
                zbuf[r, pl.ds(j * NLANE, NLANE)] = jnp.zeros((NLANE,), jnp.float32)

        bufs = [va, vb]

        def do_chunk(base, n, cv, rv, wvv, rr, src, dst):
            pltpu.sync_copy(col_h.at[pl.ds(base, n)], cv)
            pltpu.sync_copy(row_h.at[pl.ds(base, n)], rv)
            pltpu.sync_copy(w_h.at[pl.ds(base, n)], wvv)
            pltpu.sync_copy(src.at[cv], rr)

            @pl.loop(0, n // NLANE)
            def _(g):
                wvec = wvv[pl.ds(g * NLANE, NLANE)]
                for e in range(NLANE):
                    we = wvec[e]
                    r = g * NLANE + e
                    for j in range(CH // NLANE):
                        rr[r, pl.ds(j * NLANE, NLANE)] = (
                            rr[r, pl.ds(j * NLANE, NLANE)] * we)

            pltpu.sync_copy(rr, dst.at[rv], add=True)

        for t in range(1, POWERS[-1] + 1):
            src = bufs[(t + 1) % 2]
            dst = bufs[t % 2]
            for z in range(ROWS_PT // ZCH):
                pltpu.sync_copy(zbuf, dst.at[pl.ds(r0 + z * ZCH, ZCH)])
            plsc.subcore_barrier()

            @pl.loop(0, NFULL)
            def _(k):
                do_chunk(s * EPT + k * CHUNK, CHUNK,
                         colv, rowv, wv, rows, src, dst)

            do_chunk(s * EPT + NFULL * CHUNK, REM,
                     colv2, rowv2, wv2, rows2, src, dst)
            plsc.subcore_barrier()

            if t in POWERS:
                p = POWERS.index(t)
                pltpu.sync_copy(dst.at[pl.ds(r0, ROWS_PT)],
                                out_h.at[p, c, pl.ds(r0, ROWS_PT)])

    return spmm_kernel(xin, col, row, w)


BN = 1000


def _mlp(x, snaps, w1p, b1, w2, b2):
    def mlp_kernel(x_ref, s_ref, w1_ref, b1_ref, w2_ref, b2_ref, o_ref):
        xb = x_ref[...]
        fs = []
        for f in range(NF):
            for c in range(NC):
                if f == 0:
                    fs.append(xb[:, c * CH:(c + 1) * CH] - s_ref[0, c])
                elif f < NF - 1:
                    fs.append(s_ref[f - 1, c] - s_ref[f, c])
                else:
                    fs.append(s_ref[NPOW - 1, c])
        feat = jnp.concatenate(fs, axis=1)
        h = jnp.dot(feat, w1_ref[...], preferred_element_type=jnp.float32)
        h = jnp.maximum(h + b1_ref[...], 0.0)
        o_ref[...] = (jnp.dot(h, w2_ref[...], preferred_element_type=jnp.float32)
                      + b2_ref[...])

    return pl.pallas_call(
        mlp_kernel,
        grid=(N // BN,),
        in_specs=[
            pl.BlockSpec((BN, C), lambda i: (i, 0)),
            pl.BlockSpec((NPOW, NC, BN, CH), lambda i: (0, 0, i, 0)),
            pl.BlockSpec((NF * C, HID), lambda i: (0, 0)),
            pl.BlockSpec((1, HID), lambda i: (0, 0)),
            pl.BlockSpec((HID, OUT), lambda i: (0, 0)),
            pl.BlockSpec((1, OUT), lambda i: (0, 0)),
        ],
        out_specs=pl.BlockSpec((BN, OUT), lambda i: (i, 0)),
        out_shape=jax.ShapeDtypeStruct((N, OUT), jnp.float32),
    )(x, snaps, w1p, b1, w2, b2)


def kernel(x, edge_index, edge_weight, W1, b1, W2, b2):
    row = edge_index[0]
    col = edge_index[1]
    xin = x.reshape(N, NC, CH).transpose(1, 0, 2)
    snaps = _sc_spmm(xin, col, row, edge_weight)
    w1p = (W1.reshape(NC, CH, NF, HID)
             .transpose(2, 0, 1, 3)
             .reshape(NF * C, HID))
    return _mlp(x, snaps, w1p, b1.reshape(1, HID), W2, b2.reshape(1, OUT))
